# baseline (speedup 1.0000x reference)
import functools

import jax
import jax.numpy as jnp
from jax.experimental import pallas as pl
from jax.experimental.pallas import tpu as pltpu
from jax.experimental.pallas import tpu_sc as plsc

E = 8
TOP_K = 2
ALPHA = 1.702
LIMIT = 7.0
LANES = 128


def _logits_body(x_ref, rw_ref, rb_ref, g_ref):
    g_ref[...] = (
        jnp.dot(rw_ref[...], x_ref[...].T, preferred_element_type=jnp.float32)
        + rb_ref[...]
    )


def _sc_router(logits):
    T = logits.shape[1]
    info = plsc.get_sparse_core_info()
    nc, ns = info.num_cores, info.num_subcores
    nw = nc * ns
    tpw = T // nw
    mesh = plsc.VectorSubcoreMesh(core_axis_name="c", subcore_axis_name="s")

    @functools.partial(
        pl.kernel,
        mesh=mesh,
        out_type=jax.ShapeDtypeStruct((E * T,), jnp.float32),
        scratch_types=[
            pltpu.VMEM((E * tpw,), jnp.float32),
            pltpu.VMEM((E * tpw,), jnp.float32),
        ],
    )
    def body(lg_hbm, out_hbm, lg_v, out_v):
        wid = jax.lax.axis_index("s") * nc + jax.lax.axis_index("c")
        base = wid * tpw
        for e in range(E):
            pltpu.sync_copy(
                lg_hbm.at[pl.ds(e * T + base, tpw)],
                lg_v.at[pl.ds(e * tpw, tpw)],
            )
        for c in range(tpw // 16):
            les = []
            m1 = jnp.full((16,), -jnp.inf, jnp.float32)
            i1 = jnp.zeros((16,), jnp.int32)
            for e in range(E):
                le = lg_v[pl.ds(e * tpw + c * 16, 16)]
                les.append(le)
                ev = jnp.full((16,), e, jnp.int32)
                cond = le > m1
                m1 = jnp.where(cond, le, m1)
                i1 = jnp.where(cond, ev, i1)
            m2 = jnp.full((16,), -jnp.inf, jnp.float32)
            i2 = jnp.zeros((16,), jnp.int32)
            for e in range(E):
                ev = jnp.full((16,), e, jnp.int32)
                cond = (les[e] > m2) & (i1 != ev)
                m2 = jnp.where(cond, les[e], m2)
                i2 = jnp.where(cond, ev, i2)
            one = jnp.full((16,), 1.0, jnp.float32)
            w1 = one / (one + jnp.exp(m2 - m1))
            w2 = one - w1
            zero = jnp.zeros((16,), jnp.float32)
            for e in range(E):
                ev = jnp.full((16,), e, jnp.int32)
                we = jnp.where(i1 == ev, w1, zero) + jnp.where(i2 == ev, w2, zero)
                out_v[pl.ds(e * tpw + c * 16, 16)] = we
        for e in range(E):
            pltpu.sync_copy(
                out_v.at[pl.ds(e * tpw, tpw)],
                out_hbm.at[pl.ds(e * T + base, tpw)],
            )

    return body(logits.reshape(E * T))


def _moe_body(x_ref, w13_ref, bg_ref, bu_ref, w2_ref, b2_ref, wts_ref,
              out_ref, xb_ref, wgb_ref, wub_ref, w2b_ref, *, tb):
    e = pl.program_id(0)
    t = pl.program_id(1)

    @pl.when((e == 0) & (t == 0))
    def _init():
        out_ref[...] = jnp.zeros_like(out_ref)
        xb_ref[...] = x_ref[...].astype(jnp.bfloat16)

    h = w2_ref.shape[1]

    @pl.when(t == 0)
    def _cast_weights():
        wgb_ref[...] = w13_ref[0, :, :h].astype(jnp.bfloat16)
        wub_ref[...] = w13_ref[0, :, h:].astype(jnp.bfloat16)
        w2b_ref[...] = w2_ref[0].astype(jnp.bfloat16)

    x = xb_ref[pl.ds(t * tb, tb), :]
    gate = jnp.dot(x, wgb_ref[...].T, preferred_element_type=jnp.float32)
    gate = gate + bg_ref[0]
    up = jnp.dot(x, wub_ref[...].T, preferred_element_type=jnp.float32)
    up = up + bu_ref[0]
    gate = jnp.minimum(gate, LIMIT)
    up = jnp.clip(up, -LIMIT, LIMIT)
    glu = gate * jax.nn.sigmoid(ALPHA * gate)
    act = (up + 1.0) * glu
    y = jnp.dot(act.astype(jnp.bfloat16), w2b_ref[...].T,
                preferred_element_type=jnp.float32) + b2_ref[0]
    lane = jax.lax.broadcasted_iota(jnp.int32, wts_ref.shape, 1)
    wcol = jnp.sum(jnp.where(lane == e, wts_ref[...], 0.0), axis=1, keepdims=True)
    out_ref[pl.ds(t * tb, tb), :] += wcol * y


def kernel(x, router_w, router_b, w13, b13, w2, b2):
    T, H = x.shape
    I2 = w13.shape[1]
    I = I2 // 2

    rwp = jnp.zeros((LANES, H), jnp.float32).at[:E].set(router_w)
    rbp = jnp.zeros((LANES, 1), jnp.float32).at[:E, 0].set(router_b)
    logits = pl.pallas_call(
        _logits_body,
        out_shape=jax.ShapeDtypeStruct((LANES, T), jnp.float32),
    )(x, rwp, rbp)

    weights = _sc_router(logits[:E]).reshape(E, T).T

    w13r = w13.reshape(E, I, 2 * H)
    bg = b13[:, 0::2].reshape(E, 1, I)
    bu = b13[:, 1::2].reshape(E, 1, I)
    b2r = b2.reshape(E, 1, H)

    TB = 256
    grid = (E, T // TB)
    out = pl.pallas_call(
        functools.partial(_moe_body, tb=TB),
        grid=grid,
        in_specs=[
            pl.BlockSpec((T, H), lambda e, t: (0, 0)),
            pl.BlockSpec((1, I, 2 * H), lambda e, t: (e, 0, 0)),
            pl.BlockSpec((1, 1, I), lambda e, t: (e, 0, 0)),
            pl.BlockSpec((1, 1, I), lambda e, t: (e, 0, 0)),
            pl.BlockSpec((1, H, I), lambda e, t: (e, 0, 0)),
            pl.BlockSpec((1, 1, H), lambda e, t: (e, 0, 0)),
            pl.BlockSpec((TB, E), lambda e, t: (t, 0)),
        ],
        out_specs=pl.BlockSpec((T, H), lambda e, t: (0, 0)),
        out_shape=jax.ShapeDtypeStruct((T, H), jnp.float32),
        scratch_shapes=[
            pltpu.VMEM((T, H), jnp.bfloat16),
            pltpu.VMEM((I, H), jnp.bfloat16),
            pltpu.VMEM((I, H), jnp.bfloat16),
            pltpu.VMEM((H, I), jnp.bfloat16),
        ],
    )(x, w13r, bg, bu, w2, b2r, weights)
    return out


# speedup vs baseline: 1.1183x; 1.1183x over previous
import functools

import jax
import jax.numpy as jnp
from jax.experimental import pallas as pl
from jax.experimental.pallas import tpu as pltpu
from jax.experimental.pallas import tpu_sc as plsc

E = 8
TOP_K = 2
ALPHA = 1.702
LIMIT = 7.0
LANES = 128


def _logits_body(x_ref, rw_ref, rb_ref, g_ref):
    g_ref[...] = (
        jnp.dot(rw_ref[...], x_ref[...].T, preferred_element_type=jnp.float32)
        + rb_ref[...]
    )


def _sc_router(logits):
    T = logits.shape[1]
    info = plsc.get_sparse_core_info()
    nc, ns = info.num_cores, info.num_subcores
    nw = nc * ns
    tpw = T // nw
    mesh = plsc.VectorSubcoreMesh(core_axis_name="c", subcore_axis_name="s")

    @functools.partial(
        pl.kernel,
        mesh=mesh,
        out_type=jax.ShapeDtypeStruct((E * T,), jnp.float32),
        scratch_types=[
            pltpu.VMEM((E * tpw,), jnp.float32),
            pltpu.VMEM((E * tpw,), jnp.float32),
        ],
    )
    def body(lg_hbm, out_hbm, lg_v, out_v):
        wid = jax.lax.axis_index("s") * nc + jax.lax.axis_index("c")
        base = wid * tpw
        for e in range(E):
            pltpu.sync_copy(
                lg_hbm.at[pl.ds(e * T + base, tpw)],
                lg_v.at[pl.ds(e * tpw, tpw)],
            )
        for c in range(tpw // 16):
            les = []
            m1 = jnp.full((16,), -jnp.inf, jnp.float32)
            i1 = jnp.zeros((16,), jnp.int32)
            for e in range(E):
                le = lg_v[pl.ds(e * tpw + c * 16, 16)]
                les.append(le)
                ev = jnp.full((16,), e, jnp.int32)
                cond = le > m1
                m1 = jnp.where(cond, le, m1)
                i1 = jnp.where(cond, ev, i1)
            m2 = jnp.full((16,), -jnp.inf, jnp.float32)
            i2 = jnp.zeros((16,), jnp.int32)
            for e in range(E):
                ev = jnp.full((16,), e, jnp.int32)
                cond = (les[e] > m2) & (i1 != ev)
                m2 = jnp.where(cond, les[e], m2)
                i2 = jnp.where(cond, ev, i2)
            one = jnp.full((16,), 1.0, jnp.float32)
            w1 = one / (one + jnp.exp(m2 - m1))
            w2 = one - w1
            zero = jnp.zeros((16,), jnp.float32)
            for e in range(E):
                ev = jnp.full((16,), e, jnp.int32)
                we = jnp.where(i1 == ev, w1, zero) + jnp.where(i2 == ev, w2, zero)
                out_v[pl.ds(e * tpw + c * 16, 16)] = we
        for e in range(E):
            pltpu.sync_copy(
                out_v.at[pl.ds(e * tpw, tpw)],
                out_hbm.at[pl.ds(e * T + base, tpw)],
            )

    return body(logits.reshape(E * T))


def _moe_body(x_ref, w13_ref, bg_ref, bu_ref, w2_ref, b2_ref, wts_ref,
              out_ref, xb_ref, wgb_ref, wub_ref, w2b_ref, *, tb):
    e = pl.program_id(0)
    t = pl.program_id(1)

    @pl.when((e == 0) & (t == 0))
    def _init():
        out_ref[...] = jnp.zeros_like(out_ref)
        xb_ref[...] = x_ref[...].astype(jnp.bfloat16)

    h = w2_ref.shape[1]

    @pl.when(t == 0)
    def _cast_weights():
        wgb_ref[...] = w13_ref[0, :, :h].astype(jnp.bfloat16)
        wub_ref[...] = w13_ref[0, :, h:].astype(jnp.bfloat16)
        w2b_ref[...] = w2_ref[0].astype(jnp.bfloat16)

    x = xb_ref[pl.ds(t * tb, tb), :]
    gate = jnp.dot(x, wgb_ref[...].T, preferred_element_type=jnp.float32)
    gate = gate + bg_ref[0]
    up = jnp.dot(x, wub_ref[...].T, preferred_element_type=jnp.float32)
    up = up + bu_ref[0]
    gate = jnp.minimum(gate, LIMIT)
    up = jnp.clip(up, -LIMIT, LIMIT)
    glu = gate * jax.nn.sigmoid(ALPHA * gate)
    act = (up + 1.0) * glu
    y = jnp.dot(act.astype(jnp.bfloat16), w2b_ref[...].T,
                preferred_element_type=jnp.float32) + b2_ref[0]
    lane = jax.lax.broadcasted_iota(jnp.int32, wts_ref.shape, 1)
    wcol = jnp.sum(jnp.where(lane == e, wts_ref[...], 0.0), axis=1, keepdims=True)
    out_ref[pl.ds(t * tb, tb), :] += wcol * y


def kernel(x, router_w, router_b, w13, b13, w2, b2):
    T, H = x.shape
    I2 = w13.shape[1]
    I = I2 // 2

    rwp = jnp.zeros((LANES, H), jnp.float32).at[:E].set(router_w)
    rbp = jnp.zeros((LANES, 1), jnp.float32).at[:E, 0].set(router_b)
    logits = pl.pallas_call(
        _logits_body,
        out_shape=jax.ShapeDtypeStruct((LANES, T), jnp.float32),
    )(x, rwp, rbp)

    weights = _sc_router(logits[:E]).reshape(E, T).T

    w13r = w13.reshape(E, I, 2 * H)
    bg = b13[:, 0::2].reshape(E, 1, I)
    bu = b13[:, 1::2].reshape(E, 1, I)
    b2r = b2.reshape(E, 1, H)

    TB = 512
    grid = (E, T // TB)
    out = pl.pallas_call(
        functools.partial(_moe_body, tb=TB),
        grid=grid,
        in_specs=[
            pl.BlockSpec((T, H), lambda e, t: (0, 0)),
            pl.BlockSpec((1, I, 2 * H), lambda e, t: (e, 0, 0)),
            pl.BlockSpec((1, 1, I), lambda e, t: (e, 0, 0)),
            pl.BlockSpec((1, 1, I), lambda e, t: (e, 0, 0)),
            pl.BlockSpec((1, H, I), lambda e, t: (e, 0, 0)),
            pl.BlockSpec((1, 1, H), lambda e, t: (e, 0, 0)),
            pl.BlockSpec((TB, E), lambda e, t: (t, 0)),
        ],
        out_specs=pl.BlockSpec((T, H), lambda e, t: (0, 0)),
        out_shape=jax.ShapeDtypeStruct((T, H), jnp.float32),
        scratch_shapes=[
            pltpu.VMEM((T, H), jnp.bfloat16),
            pltpu.VMEM((I, H), jnp.bfloat16),
            pltpu.VMEM((I, H), jnp.bfloat16),
            pltpu.VMEM((H, I), jnp.bfloat16),
        ],
    )(x, w13r, bg, bu, w2, b2r, weights)
    return out


# speedup vs baseline: 1.1771x; 1.0526x over previous
import functools

import jax
import jax.numpy as jnp
from jax.experimental import pallas as pl
from jax.experimental.pallas import tpu as pltpu
from jax.experimental.pallas import tpu_sc as plsc

E = 8
TOP_K = 2
ALPHA = 1.702
LIMIT = 7.0
LANES = 128


def _logits_body(x_ref, rw_ref, rb_ref, g_ref):
    g_ref[...] = (
        jnp.dot(rw_ref[...], x_ref[...].T, preferred_element_type=jnp.float32)
        + rb_ref[...]
    )


def _sc_router(logits):
    T = logits.shape[1]
    info = plsc.get_sparse_core_info()
    nc, ns = info.num_cores, info.num_subcores
    nw = nc * ns
    tpw = T // nw
    mesh = plsc.VectorSubcoreMesh(core_axis_name="c", subcore_axis_name="s")

    @functools.partial(
        pl.kernel,
        mesh=mesh,
        out_type=jax.ShapeDtypeStruct((E * T,), jnp.float32),
        scratch_types=[
            pltpu.VMEM((E * tpw,), jnp.float32),
            pltpu.VMEM((E * tpw,), jnp.float32),
        ],
    )
    def body(lg_hbm, out_hbm, lg_v, out_v):
        wid = jax.lax.axis_index("s") * nc + jax.lax.axis_index("c")
        base = wid * tpw
        for e in range(E):
            pltpu.sync_copy(
                lg_hbm.at[pl.ds(e * T + base, tpw)],
                lg_v.at[pl.ds(e * tpw, tpw)],
            )
        for c in range(tpw // 16):
            les = []
            m1 = jnp.full((16,), -jnp.inf, jnp.float32)
            i1 = jnp.zeros((16,), jnp.int32)
            for e in range(E):
                le = lg_v[pl.ds(e * tpw + c * 16, 16)]
                les.append(le)
                ev = jnp.full((16,), e, jnp.int32)
                cond = le > m1
                m1 = jnp.where(cond, le, m1)
                i1 = jnp.where(cond, ev, i1)
            m2 = jnp.full((16,), -jnp.inf, jnp.float32)
            i2 = jnp.zeros((16,), jnp.int32)
            for e in range(E):
                ev = jnp.full((16,), e, jnp.int32)
                cond = (les[e] > m2) & (i1 != ev)
                m2 = jnp.where(cond, les[e], m2)
                i2 = jnp.where(cond, ev, i2)
            one = jnp.full((16,), 1.0, jnp.float32)
            w1 = one / (one + jnp.exp(m2 - m1))
            w2 = one - w1
            zero = jnp.zeros((16,), jnp.float32)
            for e in range(E):
                ev = jnp.full((16,), e, jnp.int32)
                we = jnp.where(i1 == ev, w1, zero) + jnp.where(i2 == ev, w2, zero)
                out_v[pl.ds(e * tpw + c * 16, 16)] = we
        for e in range(E):
            pltpu.sync_copy(
                out_v.at[pl.ds(e * tpw, tpw)],
                out_hbm.at[pl.ds(e * T + base, tpw)],
            )

    return body(logits.reshape(E * T))


def _moe_body(x_ref, w13_ref, bg_ref, bu_ref, w2_ref, b2_ref, wts_ref,
              out_ref, xb_ref, wgb_ref, wub_ref, w2b_ref, *, tb):
    e = pl.program_id(0)
    t = pl.program_id(1)

    @pl.when((e == 0) & (t == 0))
    def _init():
        out_ref[...] = jnp.zeros_like(out_ref)
        xb_ref[...] = x_ref[...].astype(jnp.bfloat16)

    h = w2_ref.shape[1]

    @pl.when(t == 0)
    def _cast_weights():
        wgb_ref[...] = w13_ref[0, :, :h].astype(jnp.bfloat16)
        wub_ref[...] = w13_ref[0, :, h:].astype(jnp.bfloat16)
        w2b_ref[...] = w2_ref[0].astype(jnp.bfloat16)

    x = xb_ref[pl.ds(t * tb, tb), :]
    gate = jnp.dot(x, wgb_ref[...].T, preferred_element_type=jnp.float32)
    gate = gate + bg_ref[0]
    up = jnp.dot(x, wub_ref[...].T, preferred_element_type=jnp.float32)
    up = up + bu_ref[0]
    gate = jnp.minimum(gate, LIMIT)
    up = jnp.clip(up, -LIMIT, LIMIT)
    glu = gate * jax.nn.sigmoid(ALPHA * gate)
    act = (up + 1.0) * glu
    y = jnp.dot(act.astype(jnp.bfloat16), w2b_ref[...].T,
                preferred_element_type=jnp.float32) + b2_ref[0]
    lane = jax.lax.broadcasted_iota(jnp.int32, wts_ref.shape, 1)
    wcol = jnp.sum(jnp.where(lane == e, wts_ref[...], 0.0), axis=1, keepdims=True)
    out_ref[pl.ds(t * tb, tb), :] += wcol * y


def kernel(x, router_w, router_b, w13, b13, w2, b2):
    T, H = x.shape
    I2 = w13.shape[1]
    I = I2 // 2

    rwp = jnp.zeros((LANES, H), jnp.float32).at[:E].set(router_w)
    rbp = jnp.zeros((LANES, 1), jnp.float32).at[:E, 0].set(router_b)
    logits = pl.pallas_call(
        _logits_body,
        out_shape=jax.ShapeDtypeStruct((LANES, T), jnp.float32),
    )(x, rwp, rbp)

    weights = _sc_router(logits[:E]).reshape(E, T).T

    w13r = w13.reshape(E, I, 2 * H)
    bg = b13[:, 0::2].reshape(E, 1, I)
    bu = b13[:, 1::2].reshape(E, 1, I)
    b2r = b2.reshape(E, 1, H)

    TB = 1024
    grid = (E, T // TB)
    out = pl.pallas_call(
        functools.partial(_moe_body, tb=TB),
        grid=grid,
        in_specs=[
            pl.BlockSpec((T, H), lambda e, t: (0, 0)),
            pl.BlockSpec((1, I, 2 * H), lambda e, t: (e, 0, 0)),
            pl.BlockSpec((1, 1, I), lambda e, t: (e, 0, 0)),
            pl.BlockSpec((1, 1, I), lambda e, t: (e, 0, 0)),
            pl.BlockSpec((1, H, I), lambda e, t: (e, 0, 0)),
            pl.BlockSpec((1, 1, H), lambda e, t: (e, 0, 0)),
            pl.BlockSpec((TB, E), lambda e, t: (t, 0)),
        ],
        out_specs=pl.BlockSpec((T, H), lambda e, t: (0, 0)),
        out_shape=jax.ShapeDtypeStruct((T, H), jnp.float32),
        scratch_shapes=[
            pltpu.VMEM((T, H), jnp.bfloat16),
            pltpu.VMEM((I, H), jnp.bfloat16),
            pltpu.VMEM((I, H), jnp.bfloat16),
            pltpu.VMEM((H, I), jnp.bfloat16),
        ],
    )(x, w13r, bg, bu, w2, b2r, weights)
    return out
